# baseline (speedup 1.0000x reference)
import functools

import jax
import jax.numpy as jnp
from jax import lax
from jax.experimental import pallas as pl
from jax.experimental.pallas import tpu as pltpu
from jax.experimental.pallas import tpu_sc as plsc

_NC = 2
_NS = 16
_NW = _NC * _NS

_GROUP = 128


def _embed_sc(table, idx3, *, groups_per_w, d_model):
    b_per_w = groups_per_w * _GROUP
    n_rows = _NW * b_per_w
    mesh = plsc.VectorSubcoreMesh(core_axis_name="c", subcore_axis_name="s")

    @functools.partial(
        pl.kernel,
        out_type=jax.ShapeDtypeStruct((n_rows, d_model), jnp.float32),
        mesh=mesh,
        scratch_types=[
            pltpu.VMEM((groups_per_w, _GROUP), jnp.int32),
            pltpu.VMEM((_GROUP, d_model), jnp.float32),
            pltpu.SemaphoreType.DMA,
        ],
        compiler_params=pltpu.CompilerParams(use_tc_tiling_on_sc=False),
    )
    def k(table_hbm, idx_hbm, out_hbm, idx_v, rows_v, sem):
        wid = lax.axis_index("s") * _NC + lax.axis_index("c")
        base = wid * b_per_w
        pltpu.sync_copy(idx_hbm.at[wid], idx_v)

        def body(j, carry):
            pltpu.async_copy(table_hbm.at[idx_v.at[j]], rows_v, sem).wait()
            pltpu.sync_copy(rows_v, out_hbm.at[pl.ds(base + j * _GROUP, _GROUP)])
            return carry

        lax.fori_loop(0, groups_per_w, body, 0)

    return k(table, idx3)


def kernel(token_ids, weight):
    b0, s0 = token_ids.shape
    vocab, d_model = weight.shape
    n = b0 * s0
    groups_per_w = n // (_NW * _GROUP)
    idx3 = token_ids.reshape(_NW, groups_per_w, _GROUP).astype(jnp.int32)
    out = _embed_sc(weight, idx3, groups_per_w=groups_per_w, d_model=d_model)
    return out.reshape(b0, s0, d_model)


# speedup vs baseline: 1.1161x; 1.1161x over previous
import functools

import jax
import jax.numpy as jnp
from jax import lax
from jax.experimental import pallas as pl
from jax.experimental.pallas import tpu as pltpu
from jax.experimental.pallas import tpu_sc as plsc

_NC = 2
_NS = 16
_NW = _NC * _NS

_GROUP = 128


_NBUF = 4


def _embed_sc(table, idx3, *, groups_per_w, d_model):
    b_per_w = groups_per_w * _GROUP
    n_rows = _NW * b_per_w
    mesh = plsc.VectorSubcoreMesh(core_axis_name="c", subcore_axis_name="s")
    n_main = groups_per_w // _NBUF - 1

    @functools.partial(
        pl.kernel,
        out_type=jax.ShapeDtypeStruct((n_rows, d_model), jnp.float32),
        mesh=mesh,
        scratch_types=[
            pltpu.VMEM((groups_per_w, _GROUP), jnp.int32),
            pltpu.VMEM((_NBUF, _GROUP, d_model), jnp.float32),
            pltpu.SemaphoreType.DMA((_NBUF,)),
        ],
        compiler_params=pltpu.CompilerParams(use_tc_tiling_on_sc=False),
    )
    def k(table_hbm, idx_hbm, out_hbm, idx_v, bufs, gsems):
        wid = lax.axis_index("s") * _NC + lax.axis_index("c")
        base = wid * b_per_w
        pltpu.sync_copy(idx_hbm.at[wid], idx_v)

        for b in range(_NBUF):
            pltpu.async_copy(table_hbm.at[idx_v.at[b]], bufs.at[b], gsems.at[b])

        def drain(b):
            pltpu.make_async_copy(
                table_hbm.at[pl.ds(0, _GROUP)], bufs.at[b], gsems.at[b]
            ).wait()

        def body(t, carry):
            for b in range(_NBUF):
                g = t * _NBUF + b
                drain(b)
                pltpu.sync_copy(
                    bufs.at[b], out_hbm.at[pl.ds(base + g * _GROUP, _GROUP)]
                )
                pltpu.async_copy(
                    table_hbm.at[idx_v.at[g + _NBUF]], bufs.at[b], gsems.at[b]
                )
            return carry

        lax.fori_loop(0, n_main, body, 0)

        for b in range(_NBUF):
            g = n_main * _NBUF + b
            drain(b)
            pltpu.sync_copy(bufs.at[b], out_hbm.at[pl.ds(base + g * _GROUP, _GROUP)])

    return k(table, idx3)


def kernel(token_ids, weight):
    b0, s0 = token_ids.shape
    vocab, d_model = weight.shape
    n = b0 * s0
    groups_per_w = n // (_NW * _GROUP)
    idx3 = token_ids.reshape(_NW, groups_per_w, _GROUP).astype(jnp.int32)
    out = _embed_sc(weight, idx3, groups_per_w=groups_per_w, d_model=d_model)
    return out.reshape(b0, s0, d_model)


# speedup vs baseline: 1.1162x; 1.0001x over previous
import functools

import jax
import jax.numpy as jnp
from jax import lax
from jax.experimental import pallas as pl
from jax.experimental.pallas import tpu as pltpu
from jax.experimental.pallas import tpu_sc as plsc

_NC = 2
_NS = 16
_NW = _NC * _NS

_NBUF = 4


def _embed_sc(table, idx3, *, rows_per_w, seq, d_model):
    half = seq // 2
    n_groups = 2 * rows_per_w
    n_rows = _NW * rows_per_w
    mesh = plsc.VectorSubcoreMesh(core_axis_name="c", subcore_axis_name="s")
    n_main = n_groups // _NBUF - 1

    @functools.partial(
        pl.kernel,
        out_type=jax.ShapeDtypeStruct((n_rows, seq, d_model), jnp.float32),
        mesh=mesh,
        scratch_types=[
            pltpu.VMEM((n_groups, half), jnp.int32),
            pltpu.VMEM((_NBUF, half, d_model), jnp.float32),
            pltpu.SemaphoreType.DMA((_NBUF,)),
        ],
        compiler_params=pltpu.CompilerParams(use_tc_tiling_on_sc=False),
    )
    def k(table_hbm, idx_hbm, out_hbm, idx_v, bufs, gsems):
        wid = lax.axis_index("s") * _NC + lax.axis_index("c")
        row0 = wid * rows_per_w
        pltpu.sync_copy(idx_hbm.at[wid], idx_v)

        for b in range(_NBUF):
            pltpu.async_copy(table_hbm.at[idx_v.at[b]], bufs.at[b], gsems.at[b])

        def drain_store(b, g):
            pltpu.make_async_copy(
                table_hbm.at[pl.ds(0, half)], bufs.at[b], gsems.at[b]
            ).wait()
            pltpu.sync_copy(
                bufs.at[b],
                out_hbm.at[row0 + g // 2, pl.ds((g % 2) * half, half)],
            )

        def body(t, carry):
            for b in range(_NBUF):
                g = t * _NBUF + b
                drain_store(b, g)
                pltpu.async_copy(
                    table_hbm.at[idx_v.at[g + _NBUF]], bufs.at[b], gsems.at[b]
                )
            return carry

        lax.fori_loop(0, n_main, body, 0)

        for b in range(_NBUF):
            drain_store(b, n_main * _NBUF + b)

    return k(table, idx3)


def kernel(token_ids, weight):
    b0, s0 = token_ids.shape
    vocab, d_model = weight.shape
    rows_per_w = b0 // _NW
    idx3 = token_ids.reshape(_NW, 2 * rows_per_w, s0 // 2).astype(jnp.int32)
    return _embed_sc(
        weight, idx3, rows_per_w=rows_per_w, seq=s0, d_model=d_model
    )


# speedup vs baseline: 1.4821x; 1.3278x over previous
import functools

import jax
import jax.numpy as jnp
from jax import lax
from jax.experimental import pallas as pl
from jax.experimental.pallas import tpu as pltpu
from jax.experimental.pallas import tpu_sc as plsc

_NC = 2
_NS = 16
_NW = _NC * _NS

_NBUF = 4


def _embed_sc(table, idx3, *, rows_per_w, seq, d_model):
    half = seq // 2
    n_groups = 2 * rows_per_w
    n_rows = _NW * rows_per_w
    mesh = plsc.VectorSubcoreMesh(core_axis_name="c", subcore_axis_name="s")
    n_main = n_groups // _NBUF - 1

    @functools.partial(
        pl.kernel,
        out_type=jax.ShapeDtypeStruct((n_rows, seq, 2 * d_model), jnp.float32),
        mesh=mesh,
        scratch_types=[
            pltpu.VMEM((n_groups, half), jnp.int32),
            pltpu.VMEM((_NBUF, half, d_model), jnp.float32),
            pltpu.SemaphoreType.DMA((_NBUF,)),
        ],
        compiler_params=pltpu.CompilerParams(use_tc_tiling_on_sc=False),
    )
    def k(table_hbm, idx_hbm, out_hbm, idx_v, bufs, gsems):
        wid = lax.axis_index("s") * _NC + lax.axis_index("c")
        row0 = wid * rows_per_w
        pltpu.sync_copy(idx_hbm.at[wid], idx_v)

        for b in range(_NBUF):
            pltpu.async_copy(table_hbm.at[idx_v.at[b]], bufs.at[b], gsems.at[b])

        def drain_store(b, g):
            pltpu.make_async_copy(
                table_hbm.at[pl.ds(0, half)], bufs.at[b], gsems.at[b]
            ).wait()
            pltpu.sync_copy(
                bufs.at[b],
                out_hbm.at[
                    row0 + g // 2, pl.ds((g % 2) * half, half), pl.ds(0, d_model)
                ],
            )

        def body(t, carry):
            for b in range(_NBUF):
                g = t * _NBUF + b
                drain_store(b, g)
                pltpu.async_copy(
                    table_hbm.at[idx_v.at[g + _NBUF]], bufs.at[b], gsems.at[b]
                )
            return carry

        lax.fori_loop(0, n_main, body, 0)

        for b in range(_NBUF):
            drain_store(b, n_main * _NBUF + b)

    return k(table, idx3)


def kernel(token_ids, weight):
    b0, s0 = token_ids.shape
    vocab, d_model = weight.shape
    rows_per_w = b0 // _NW
    idx3 = token_ids.reshape(_NW, 2 * rows_per_w, s0 // 2).astype(jnp.int32)
    out2 = _embed_sc(
        weight, idx3, rows_per_w=rows_per_w, seq=s0, d_model=d_model
    )
    return out2[:, :, :d_model]
